# Initial kernel scaffold; baseline (speedup 1.0000x reference)
import jax
import jax.numpy as jnp
from jax.experimental import pallas as pl


def kernel(elsa_emb, text_emb, e_w1, e_b1, e_g1, e_be1, e_w2, e_b2, e_g2, e_be2, t_w1, t_b1, t_g1, t_be1, t_w2, t_b2, t_g2, t_be2, f_w, f_b, f_g, f_be, enc_w, enc_b, dec_w, dec_b):
    raise NotImplementedError("write your pallas kernel here")


# speedup vs baseline: 1.6680x; 1.6680x over previous
import functools

import jax
import jax.numpy as jnp
from jax import lax
from jax.experimental import pallas as pl
from jax.experimental.pallas import tpu as pltpu

K = 32

_ERFC_P = (+2.326819970068386e-2, -1.387039388740657e-1, +3.687424674597105e-1,
           -5.824733027278666e-1, +6.210004621745983e-1, -4.944515323274145e-1,
           +3.404879937665872e-1, -2.741127028184656e-1, +5.638259427386472e-1)
_ERFC_R = (-1.047766399936249e+1, +1.297719955372516e+1, -7.495518717768503e+0,
           +2.921019019210786e+0, -1.015265279202700e+0, +4.218463358204948e-1,
           -2.820767439740514e-1, +5.641895067754075e-1)
_ERF_T = (+7.853861353153693e-5, -8.010193625184903e-4, +5.188327685732524e-3,
          -2.685381193529856e-2, +1.128358514861418e-1, -3.761262582423300e-1,
          +1.128379165726710e+0)


def _poly(y, coefs):
    acc = jnp.full_like(y, coefs[0])
    for c in coefs[1:]:
        acc = acc * y + c
    return acc


def _erfc(x):
    ax = jnp.abs(x)
    z = jnp.exp(-x * x)
    q = 1.0 / ax
    y = q * q
    p = jnp.where(ax < 2.0, _poly(y, _ERFC_P), _poly(y, _ERFC_R))
    yv = z * q * p
    yv = jnp.where(z == 0.0, 0.0, yv)
    big = jnp.where(x < 0.0, 2.0 - yv, yv)
    erf_small = x * _poly(x * x, _ERF_T)
    return jnp.where(ax < 1.0, 1.0 - erf_small, big)


def _sum_rows(x):
    w = x.shape[-1]
    if w == 256:
        x = x[:, :128] + x[:, 128:]
        w = 128
    assert w == 128
    r = x.shape[0]
    x3 = x.reshape(r, 16, 8)
    acc = x3[:, 0, :]
    for a in range(1, 16):
        acc = acc + x3[:, a, :]
    t = acc[:, :4] + acc[:, 4:]
    t = t[:, :2] + t[:, 2:]
    return t[:, 0:1] + t[:, 1:2]


def _ln(x, g, b):
    inv_w = jnp.float32(1.0 / x.shape[-1])
    m = _sum_rows(x) * inv_w
    v = _sum_rows((x - m) ** 2) * inv_w
    return (x - m) / jnp.sqrt(v + 1e-5) * g + b


def _dot_t(x, w):
    return lax.dot_general(x, w, (((1,), (1,)), ((), ())),
                           preferred_element_type=jnp.float32)


def _proj(x, w1, b1, g1, be1, w2, b2, g2, be2):
    h = _dot_t(x, w1) + b1
    h = _ln(h, g1, be1)
    h = 0.5 * h * _erfc(-h * jnp.float32(0.7071067811865476))
    h = _dot_t(h, w2) + b2
    return _ln(h, g2, be2)


def _body(elsa, text,
          e_w1, e_b1, e_g1, e_be1, e_w2, e_b2, e_g2, e_be2,
          t_w1, t_b1, t_g1, t_be1, t_w2, t_b2, t_g2, t_be2,
          f_w, f_b, f_g, f_be,
          enc_w, enc_b, dec_w, dec_b,
          recon_out, hsp_out, comb_out, idx_out,
          hpre_ref):
    ep = _proj(elsa[...], e_w1[...], e_b1[...], e_g1[...], e_be1[...],
               e_w2[...], e_b2[...], e_g2[...], e_be2[...])
    tp = _proj(text[...], t_w1[...], t_b1[...], t_g1[...], t_be1[...],
               t_w2[...], t_b2[...], t_g2[...], t_be2[...])
    combined = jnp.concatenate([ep, tp], axis=-1)
    combined = _ln(_dot_t(combined, f_w[...]) + f_b[...], f_g[...], f_be[...])
    nrm = jnp.maximum(jnp.sqrt(_sum_rows(combined * combined)), 1e-12)
    combined = combined / nrm
    comb_out[...] = combined

    hpre = _dot_t(combined, enc_w[...]) + enc_b[...]
    hpre_ref[...] = hpre

    r, h = hpre.shape
    iot = lax.broadcasted_iota(jnp.int32, (r, h), 1)
    kcol = lax.broadcasted_iota(jnp.int32, (r, K), 1)
    neg_inf = jnp.float32(-jnp.inf)

    def step(k, carry):
        x, idx = carry
        m = jnp.max(x, axis=1, keepdims=True)
        j = jnp.min(jnp.where(x == m, iot, h), axis=1, keepdims=True)
        idx = jnp.where(kcol == k, j, idx)
        x = jnp.where((x == m) & (iot == j), neg_inf, x)
        return x, idx

    x0 = hpre
    idx0 = jnp.zeros((r, K), jnp.int32)
    xf, idx = lax.fori_loop(0, K, step, (x0, idx0))
    idx_out[...] = idx

    hsp = jnp.where(xf == neg_inf, jnp.maximum(hpre, 0.0), 0.0)
    hsp_out[...] = hsp

    recon_out[...] = _dot_t(hsp, dec_w[...]) + dec_b[...]


def kernel(elsa_emb, text_emb,
           e_w1, e_b1, e_g1, e_be1, e_w2, e_b2, e_g2, e_be2,
           t_w1, t_b1, t_g1, t_be1, t_w2, t_b2, t_g2, t_be2,
           f_w, f_b, f_g, f_be,
           enc_w, enc_b, dec_w, dec_b):
    B = elsa_emb.shape[0]
    H, C = enc_w.shape
    R = min(64, B)
    grid = (B // R,)

    def row2d(a):
        return a.reshape(1, -1)

    weights = (e_w1, row2d(e_b1), row2d(e_g1), row2d(e_be1),
               e_w2, row2d(e_b2), row2d(e_g2), row2d(e_be2),
               t_w1, row2d(t_b1), row2d(t_g1), row2d(t_be1),
               t_w2, row2d(t_b2), row2d(t_g2), row2d(t_be2),
               f_w, row2d(f_b), row2d(f_g), row2d(f_be),
               enc_w, row2d(enc_b), dec_w, row2d(dec_b))

    def tile_spec(shape):
        return pl.BlockSpec((R, shape[1]), lambda i: (i, 0))

    def const_spec(a):
        return pl.BlockSpec(a.shape, lambda i: (0,) * a.ndim)

    out_shapes = (
        jax.ShapeDtypeStruct((B, C), jnp.float32),
        jax.ShapeDtypeStruct((B, H), jnp.float32),
        jax.ShapeDtypeStruct((B, C), jnp.float32),
        jax.ShapeDtypeStruct((B, K), jnp.int32),
    )
    out_specs = (
        pl.BlockSpec((R, C), lambda i: (i, 0)),
        pl.BlockSpec((R, H), lambda i: (i, 0)),
        pl.BlockSpec((R, C), lambda i: (i, 0)),
        pl.BlockSpec((R, K), lambda i: (i, 0)),
    )

    fn = pl.pallas_call(
        _body,
        grid=grid,
        in_specs=[tile_spec(elsa_emb.shape), tile_spec(text_emb.shape)]
                 + [const_spec(w) for w in weights],
        out_specs=out_specs,
        out_shape=out_shapes,
        scratch_shapes=[pltpu.VMEM((R, H), jnp.float32)],
    )
    recon, hsp, comb, idx = fn(elsa_emb, text_emb, *weights)
    return (recon, hsp, comb, idx)


# speedup vs baseline: 1.9210x; 1.1517x over previous
import functools

import jax
import jax.numpy as jnp
from jax import lax
from jax.experimental import pallas as pl
from jax.experimental.pallas import tpu as pltpu

K = 32

_ERFC_P = (+2.326819970068386e-2, -1.387039388740657e-1, +3.687424674597105e-1,
           -5.824733027278666e-1, +6.210004621745983e-1, -4.944515323274145e-1,
           +3.404879937665872e-1, -2.741127028184656e-1, +5.638259427386472e-1)
_ERFC_R = (-1.047766399936249e+1, +1.297719955372516e+1, -7.495518717768503e+0,
           +2.921019019210786e+0, -1.015265279202700e+0, +4.218463358204948e-1,
           -2.820767439740514e-1, +5.641895067754075e-1)
_ERF_T = (+7.853861353153693e-5, -8.010193625184903e-4, +5.188327685732524e-3,
          -2.685381193529856e-2, +1.128358514861418e-1, -3.761262582423300e-1,
          +1.128379165726710e+0)


def _poly(y, coefs):
    acc = jnp.full_like(y, coefs[0])
    for c in coefs[1:]:
        acc = acc * y + c
    return acc


def _erfc(x):
    ax = jnp.abs(x)
    z = jnp.exp(-x * x)
    q = 1.0 / ax
    y = q * q
    p = jnp.where(ax < 2.0, _poly(y, _ERFC_P), _poly(y, _ERFC_R))
    yv = z * q * p
    yv = jnp.where(z == 0.0, 0.0, yv)
    big = jnp.where(x < 0.0, 2.0 - yv, yv)
    erf_small = x * _poly(x * x, _ERF_T)
    return jnp.where(ax < 1.0, 1.0 - erf_small, big)


def _sum_rows(x):
    w = x.shape[-1]
    if w == 256:
        x = x[:, :128] + x[:, 128:]
        w = 128
    assert w == 128
    r = x.shape[0]
    x3 = x.reshape(r, 16, 8)
    acc = x3[:, 0, :]
    for a in range(1, 16):
        acc = acc + x3[:, a, :]
    t = acc[:, :4] + acc[:, 4:]
    t = t[:, :2] + t[:, 2:]
    return t[:, 0:1] + t[:, 1:2]


def _ln(x, g, b):
    inv_w = jnp.float32(1.0 / x.shape[-1])
    m = _sum_rows(x) * inv_w
    v = _sum_rows((x - m) ** 2) * inv_w
    return (x - m) / jnp.sqrt(v + 1e-5) * g + b


def _dot_t(x, w):
    return lax.dot_general(x, w, (((1,), (1,)), ((), ())),
                           preferred_element_type=jnp.float32)


def _proj(x, w1, b1, g1, be1, w2, b2, g2, be2):
    h = _dot_t(x, w1) + b1
    h = _ln(h, g1, be1)
    h = 0.5 * h * _erfc(-h * jnp.float32(0.7071067811865476))
    h = _dot_t(h, w2) + b2
    return _ln(h, g2, be2)


def _body(elsa, text,
          e_w1, e_b1, e_g1, e_be1, e_w2, e_b2, e_g2, e_be2,
          t_w1, t_b1, t_g1, t_be1, t_w2, t_b2, t_g2, t_be2,
          f_w, f_b, f_g, f_be,
          enc_w, enc_b, dec_w, dec_b,
          recon_out, hsp_out, comb_out, idx_out,
          hpre_ref):
    ep = _proj(elsa[...], e_w1[...], e_b1[...], e_g1[...], e_be1[...],
               e_w2[...], e_b2[...], e_g2[...], e_be2[...])
    tp = _proj(text[...], t_w1[...], t_b1[...], t_g1[...], t_be1[...],
               t_w2[...], t_b2[...], t_g2[...], t_be2[...])
    combined = jnp.concatenate([ep, tp], axis=-1)
    combined = _ln(_dot_t(combined, f_w[...]) + f_b[...], f_g[...], f_be[...])
    nrm = jnp.maximum(jnp.sqrt(_sum_rows(combined * combined)), 1e-12)
    combined = combined / nrm
    comb_out[...] = combined

    hpre = _dot_t(combined, enc_w[...]) + enc_b[...]
    hpre_ref[...] = hpre

    r, h = hpre.shape
    iot = lax.broadcasted_iota(jnp.int32, (r, h), 1)
    kcol = lax.broadcasted_iota(jnp.int32, (r, K), 1)
    neg_inf = jnp.float32(-jnp.inf)

    def step(k, carry):
        x, idx = carry
        m = jnp.max(x, axis=1, keepdims=True)
        j = jnp.min(jnp.where(x == m, iot, h), axis=1, keepdims=True)
        idx = jnp.where(kcol == k, j, idx)
        x = jnp.where(iot == j, neg_inf, x)
        return x, idx

    x0 = hpre
    idx0 = jnp.zeros((r, K), jnp.int32)
    xf, idx = lax.fori_loop(0, K, step, (x0, idx0))
    idx_out[...] = idx

    hsp = jnp.where(xf == neg_inf, jnp.maximum(hpre, 0.0), 0.0)
    hsp_out[...] = hsp

    recon_out[...] = _dot_t(hsp, dec_w[...]) + dec_b[...]


def kernel(elsa_emb, text_emb,
           e_w1, e_b1, e_g1, e_be1, e_w2, e_b2, e_g2, e_be2,
           t_w1, t_b1, t_g1, t_be1, t_w2, t_b2, t_g2, t_be2,
           f_w, f_b, f_g, f_be,
           enc_w, enc_b, dec_w, dec_b):
    B = elsa_emb.shape[0]
    H, C = enc_w.shape
    R = min(64, B)
    grid = (B // R,)

    def row2d(a):
        return a.reshape(1, -1)

    weights = (e_w1, row2d(e_b1), row2d(e_g1), row2d(e_be1),
               e_w2, row2d(e_b2), row2d(e_g2), row2d(e_be2),
               t_w1, row2d(t_b1), row2d(t_g1), row2d(t_be1),
               t_w2, row2d(t_b2), row2d(t_g2), row2d(t_be2),
               f_w, row2d(f_b), row2d(f_g), row2d(f_be),
               enc_w, row2d(enc_b), dec_w, row2d(dec_b))

    def tile_spec(shape):
        return pl.BlockSpec((R, shape[1]), lambda i: (i, 0))

    def const_spec(a):
        return pl.BlockSpec(a.shape, lambda i: (0,) * a.ndim)

    out_shapes = (
        jax.ShapeDtypeStruct((B, C), jnp.float32),
        jax.ShapeDtypeStruct((B, H), jnp.float32),
        jax.ShapeDtypeStruct((B, C), jnp.float32),
        jax.ShapeDtypeStruct((B, K), jnp.int32),
    )
    out_specs = (
        pl.BlockSpec((R, C), lambda i: (i, 0)),
        pl.BlockSpec((R, H), lambda i: (i, 0)),
        pl.BlockSpec((R, C), lambda i: (i, 0)),
        pl.BlockSpec((R, K), lambda i: (i, 0)),
    )

    fn = pl.pallas_call(
        _body,
        grid=grid,
        in_specs=[tile_spec(elsa_emb.shape), tile_spec(text_emb.shape)]
                 + [const_spec(w) for w in weights],
        out_specs=out_specs,
        out_shape=out_shapes,
        scratch_shapes=[pltpu.VMEM((R, H), jnp.float32)],
    )
    recon, hsp, comb, idx = fn(elsa_emb, text_emb, *weights)
    return (recon, hsp, comb, idx)
